# Initial kernel scaffold; baseline (speedup 1.0000x reference)
import jax
import jax.numpy as jnp
from jax import lax
from jax.experimental import pallas as pl
from jax.experimental.pallas import tpu as pltpu
from jax.experimental.pallas import tpu_sc as plsc

N = 10000
E = 320000
D = 128

NC = 2
NS = 16
L = 16
NW = NC * NS
E_PER_W = E // NW
C = 80
NCHUNK = E_PER_W // C
ROWS_PER_TILE = N // NS


def _spmm_body(feat_hbm, einfo_hbm, out0_hbm, out1_hbm,
               acc_sh, ebuf, rows_v, zrow_v, sem):
  cid = lax.axis_index("c")
  sid = lax.axis_index("s")
  wid = sid * NC + cid

  def _zlane(i, _):
    zrow_v[0, pl.ds(i * L, L)] = jnp.zeros((L,), jnp.float32)
    return 0
  lax.fori_loop(0, D // L, _zlane, 0)

  def _zrow(r, _):
    pltpu.sync_copy(zrow_v, acc_sh.at[pl.ds(sid * ROWS_PER_TILE + r, 1)])
    return 0
  lax.fori_loop(0, ROWS_PER_TILE, _zrow, 0)
  plsc.subcore_barrier()

  def _chunk(ci, _):
    pltpu.sync_copy(einfo_hbm.at[wid, ci], ebuf)
    pltpu.async_copy(feat_hbm.at[ebuf.at[0]], rows_v, sem).wait()

    def _edge(e, _):
      wbits = plsc.load_gather(
          ebuf, [jnp.full((L,), 2, jnp.int32), jnp.full((L,), e, jnp.int32)])
      wvec = plsc.bitcast(wbits, jnp.float32)
      for k in range(D // L):
        rows_v[e, pl.ds(k * L, L)] = rows_v[e, pl.ds(k * L, L)] * wvec
      return 0
    lax.fori_loop(0, C, _edge, 0)

    pltpu.sync_copy(rows_v, acc_sh.at[ebuf.at[1]], add=True)
    return 0
  lax.fori_loop(0, NCHUNK, _chunk, 0)

  plsc.subcore_barrier()

  r0 = sid * ROWS_PER_TILE

  @pl.when(cid == 0)
  def _():
    pltpu.sync_copy(acc_sh.at[pl.ds(r0, ROWS_PER_TILE)],
                    out0_hbm.at[pl.ds(r0, ROWS_PER_TILE)])

  @pl.when(cid == 1)
  def _():
    pltpu.sync_copy(acc_sh.at[pl.ds(r0, ROWS_PER_TILE)],
                    out1_hbm.at[pl.ds(r0, ROWS_PER_TILE)])


_spmm_sc = pl.kernel(
    _spmm_body,
    out_type=(jax.ShapeDtypeStruct((N, D), jnp.float32),
              jax.ShapeDtypeStruct((N, D), jnp.float32)),
    mesh=plsc.VectorSubcoreMesh(core_axis_name="c", subcore_axis_name="s",
                                num_cores=NC, num_subcores=NS),
    scratch_types=[
        pltpu.VMEM_SHARED((N, D), jnp.float32),
        pltpu.VMEM((3, C), jnp.int32),
        pltpu.VMEM((C, D), jnp.float32),
        pltpu.VMEM((1, D), jnp.float32),
        pltpu.SemaphoreType.DMA,
    ],
)


def _transform_body(x_ref, p0_ref, p1_ref, w0t_ref, w1t_ref, v0_ref, v1_ref,
                    out_ref):
  x = x_ref[...]
  h1 = p0_ref[...] + p1_ref[...]

  def _branch(h, wt_ref, v_ref):
    b = v_ref[0:1, :]
    scale = v_ref[1:2, :]
    offset = v_ref[2:3, :]
    y = jnp.maximum(
        jnp.dot(h, wt_ref[...], preferred_element_type=jnp.float32) + b, 0.0)
    mean = jnp.mean(y, axis=1, keepdims=True)
    yc = y - mean
    var = jnp.mean(yc * yc, axis=1, keepdims=True) + 1e-9
    return yc * scale * lax.rsqrt(var) + offset

  out_ref[...] = (_branch(x, w0t_ref, v0_ref) +
                  _branch(h1, w1t_ref, v1_ref))


def _transform_tc(feat_in, p0, p1, w0t, w1t, v0, v1):
  blk = 1000
  grid = N // blk
  return pl.pallas_call(
      _transform_body,
      out_shape=jax.ShapeDtypeStruct((N, D), jnp.float32),
      grid=(grid,),
      in_specs=[
          pl.BlockSpec((blk, D), lambda i: (i, 0)),
          pl.BlockSpec((blk, D), lambda i: (i, 0)),
          pl.BlockSpec((blk, D), lambda i: (i, 0)),
          pl.BlockSpec((D, D), lambda i: (0, 0)),
          pl.BlockSpec((D, D), lambda i: (0, 0)),
          pl.BlockSpec((8, D), lambda i: (0, 0)),
          pl.BlockSpec((8, D), lambda i: (0, 0)),
      ],
      out_specs=pl.BlockSpec((blk, D), lambda i: (i, 0)),
  )(feat_in, p0, p1, w0t, w1t, v0, v1)


@jax.jit
def kernel(feat_in, edge_index, edge_weight, W0, W1, b0, b1, scale0, scale1,
           offset0, offset1):
  dst = edge_index[0]
  src = edge_index[1]
  einfo = jnp.stack([src, dst,
                     lax.bitcast_convert_type(edge_weight, jnp.int32)])
  einfo = einfo.reshape(3, NW, NCHUNK, C).transpose(1, 2, 0, 3)

  p0, p1 = _spmm_sc(feat_in, einfo)

  def _pack(b, s, o):
    v = jnp.stack([b, s, o])
    return jnp.concatenate([v, jnp.zeros((5, D), jnp.float32)], axis=0)

  out = _transform_tc(feat_in, p0, p1, W0.T, W1.T,
                      _pack(b0, scale0, offset0), _pack(b1, scale1, offset1))
  return out


# speedup vs baseline: 4.6793x; 4.6793x over previous
import jax
import jax.numpy as jnp
from jax import lax
from jax.experimental import pallas as pl
from jax.experimental.pallas import tpu as pltpu
from jax.experimental.pallas import tpu_sc as plsc

N = 10000
E = 320000
D = 128

NC = 2
NS = 16
L = 16
NW = NC * NS
E_PER_W = E // NW
C = 80
NCHUNK = E_PER_W // C
RB = 16
NRB = N // RB
RB_PER_TILE = (NRB + NS - 1) // NS


_GDN = lax.GatherDimensionNumbers(
    offset_dims=(), collapsed_slice_dims=(0,), start_index_map=(0,))


def _lane_bcast(v, j):
  idx = jnp.full((L, 1), j, jnp.int32)
  return lax.gather(v, idx, _GDN, (1,),
                    mode=lax.GatherScatterMode.PROMISE_IN_BOUNDS)


def _spmm_body(feat_hbm, einfo_hbm, w_hbm, out0_hbm, out1_hbm,
               acc_sh, ebuf, wbuf, rows_v, zrow_v, sem):
  cid = lax.axis_index("c")
  sid = lax.axis_index("s")
  wid = sid * NC + cid

  def _zlane(i, _):
    r = i // (D // L)
    k = i % (D // L)
    zrow_v[r, pl.ds(k * L, L)] = jnp.zeros((L,), jnp.float32)
    return 0
  lax.fori_loop(0, RB * (D // L), _zlane, 0)

  def _zblk(j, _):
    b = j * NS + sid

    @pl.when(b < NRB)
    def _():
      pltpu.sync_copy(zrow_v, acc_sh.at[pl.ds(b * RB, RB)])
    return 0
  lax.fori_loop(0, RB_PER_TILE, _zblk, 0)
  plsc.subcore_barrier()

  def _chunk(ci, _):
    pltpu.sync_copy(einfo_hbm.at[wid, ci], ebuf)
    pltpu.sync_copy(w_hbm.at[wid, ci], wbuf)
    pltpu.async_copy(feat_hbm.at[ebuf.at[0]], rows_v, sem).wait()

    def _group(g, _):
      wvec = wbuf[pl.ds(g * L, L)]
      for j in range(L):
        wb = _lane_bcast(wvec, j)
        e = g * L + j
        for k in range(D // L):
          rows_v[e, pl.ds(k * L, L)] = rows_v[e, pl.ds(k * L, L)] * wb
      return 0
    lax.fori_loop(0, C // L, _group, 0)

    pltpu.sync_copy(rows_v, acc_sh.at[ebuf.at[1]], add=True)
    return 0
  lax.fori_loop(0, NCHUNK, _chunk, 0)

  plsc.subcore_barrier()

  def _wblk(j, _):
    b = j * NS + sid

    @pl.when(b < NRB)
    def _():
      @pl.when(cid == 0)
      def _():
        pltpu.sync_copy(acc_sh.at[pl.ds(b * RB, RB)],
                        out0_hbm.at[pl.ds(b * RB, RB)])

      @pl.when(cid == 1)
      def _():
        pltpu.sync_copy(acc_sh.at[pl.ds(b * RB, RB)],
                        out1_hbm.at[pl.ds(b * RB, RB)])
    return 0
  lax.fori_loop(0, RB_PER_TILE, _wblk, 0)


_spmm_sc = pl.kernel(
    _spmm_body,
    out_type=(jax.ShapeDtypeStruct((N, D), jnp.float32),
              jax.ShapeDtypeStruct((N, D), jnp.float32)),
    mesh=plsc.VectorSubcoreMesh(core_axis_name="c", subcore_axis_name="s",
                                num_cores=NC, num_subcores=NS),
    scratch_types=[
        pltpu.VMEM_SHARED((N, D), jnp.float32),
        pltpu.VMEM((2, C), jnp.int32),
        pltpu.VMEM((C,), jnp.float32),
        pltpu.VMEM((C, D), jnp.float32),
        pltpu.VMEM((RB, D), jnp.float32),
        pltpu.SemaphoreType.DMA,
    ],
)


def _transform_body(x_ref, p0_ref, p1_ref, w0t_ref, w1t_ref, v0_ref, v1_ref,
                    out_ref):
  x = x_ref[...]
  h1 = p0_ref[...] + p1_ref[...]

  def _branch(h, wt_ref, v_ref):
    b = v_ref[0:1, :]
    scale = v_ref[1:2, :]
    offset = v_ref[2:3, :]
    y = jnp.maximum(
        jnp.dot(h, wt_ref[...], preferred_element_type=jnp.float32) + b, 0.0)
    mean = jnp.mean(y, axis=1, keepdims=True)
    yc = y - mean
    var = jnp.mean(yc * yc, axis=1, keepdims=True) + 1e-9
    return yc * scale * lax.rsqrt(var) + offset

  out_ref[...] = (_branch(x, w0t_ref, v0_ref) +
                  _branch(h1, w1t_ref, v1_ref))


def _transform_tc(feat_in, p0, p1, w0t, w1t, v0, v1):
  blk = 1000
  grid = N // blk
  return pl.pallas_call(
      _transform_body,
      out_shape=jax.ShapeDtypeStruct((N, D), jnp.float32),
      grid=(grid,),
      in_specs=[
          pl.BlockSpec((blk, D), lambda i: (i, 0)),
          pl.BlockSpec((blk, D), lambda i: (i, 0)),
          pl.BlockSpec((blk, D), lambda i: (i, 0)),
          pl.BlockSpec((D, D), lambda i: (0, 0)),
          pl.BlockSpec((D, D), lambda i: (0, 0)),
          pl.BlockSpec((8, D), lambda i: (0, 0)),
          pl.BlockSpec((8, D), lambda i: (0, 0)),
      ],
      out_specs=pl.BlockSpec((blk, D), lambda i: (i, 0)),
  )(feat_in, p0, p1, w0t, w1t, v0, v1)


@jax.jit
def kernel(feat_in, edge_index, edge_weight, W0, W1, b0, b1, scale0, scale1,
           offset0, offset1):
  dst = edge_index[0]
  src = edge_index[1]
  einfo = jnp.stack([src, dst]).reshape(2, NW, NCHUNK, C).transpose(1, 2, 0, 3)
  wgt = edge_weight.reshape(NW, NCHUNK, C)

  p0, p1 = _spmm_sc(feat_in, einfo, wgt)

  def _pack(b, s, o):
    v = jnp.stack([b, s, o])
    return jnp.concatenate([v, jnp.zeros((5, D), jnp.float32)], axis=0)

  out = _transform_tc(feat_in, p0, p1, W0.T, W1.T,
                      _pack(b0, scale0, offset0), _pack(b1, scale1, offset1))
  return out


# speedup vs baseline: 9.7076x; 2.0746x over previous
import jax
import jax.numpy as jnp
from jax import lax
from jax.experimental import pallas as pl
from jax.experimental.pallas import tpu as pltpu
from jax.experimental.pallas import tpu_sc as plsc

N = 10000
E = 320000
D = 128

NC = 2
NS = 16
L = 16
NW = NC * NS
C = 112
NCHUNK = 90
E_PER_W = NCHUNK * C
E_PAD = NW * E_PER_W
RB = 16
NRB = N // RB
RB_PER_TILE = (NRB + NS - 1) // NS

_GDN = lax.GatherDimensionNumbers(
    offset_dims=(), collapsed_slice_dims=(0,), start_index_map=(0,))


def _lane_bcast(v, j):
  idx = jnp.full((L, 1), j, jnp.int32)
  return lax.gather(v, idx, _GDN, (1,),
                    mode=lax.GatherScatterMode.PROMISE_IN_BOUNDS)


def _spmm_body(feat_hbm, einfo_hbm, w_hbm, out0_hbm, out1_hbm,
               acc_sh, eb0, eb1, wb0, wb1, db0, db1, rw0, rw1, zrow_v,
               sem_e0, sem_e1, sem_w0, sem_w1, sem_g0, sem_g1,
               sem_t0, sem_t1):
  cid = lax.axis_index("c")
  sid = lax.axis_index("s")
  wid = sid * NC + cid

  def _zlane(i, _):
    r = i // (D // L)
    k = i % (D // L)
    zrow_v[r, pl.ds(k * L, L)] = jnp.zeros((L,), jnp.float32)
    return 0
  lax.fori_loop(0, RB * (D // L), _zlane, 0)

  def _zblk(j, _):
    b = j * NS + sid

    @pl.when(b < NRB)
    def _():
      pltpu.sync_copy(zrow_v, acc_sh.at[pl.ds(b * RB, RB)])
    return 0
  lax.fori_loop(0, RB_PER_TILE, _zblk, 0)
  plsc.subcore_barrier()

  def _scale(rw, wb):
    def _group(g, _):
      wvec = wb[pl.ds(g * L, L)]
      for j in range(L):
        wj = _lane_bcast(wvec, j)
        e = g * L + j
        for k in range(D // L):
          rw[e, pl.ds(k * L, L)] = rw[e, pl.ds(k * L, L)] * wj
      return 0
    lax.fori_loop(0, C // L, _group, 0)

  def _half(c, ebp, wbp, dbp, rwp, sem_ep, sem_wp, sem_gp, sem_tp,
            ebq, dbq, rwq, sem_eq, sem_gq, sem_tq):
    pltpu.make_async_copy(feat_hbm.at[ebp.at[0]], rwp, sem_gp).wait()
    for t in range(C // L):
      dbp[pl.ds(t * L, L)] = ebp[1, pl.ds(t * L, L)]

    @pl.when(c + 2 < NCHUNK)
    def _():
      pltpu.async_copy(einfo_hbm.at[wid, c + 2], ebp, sem_ep)

    @pl.when(c + 1 < NCHUNK)
    def _():
      pltpu.make_async_copy(einfo_hbm.at[wid, 0], ebq, sem_eq).wait()

      @pl.when(c >= 1)
      def _():
        pltpu.make_async_copy(rwq, acc_sh.at[dbq], sem_tq).wait()
      pltpu.async_copy(feat_hbm.at[ebq.at[0]], rwq, sem_gq)

    pltpu.make_async_copy(w_hbm.at[wid, 0], wbp, sem_wp).wait()
    _scale(rwp, wbp)

    @pl.when(c + 2 < NCHUNK)
    def _():
      pltpu.async_copy(w_hbm.at[wid, c + 2], wbp, sem_wp)

    pltpu.async_copy(rwp, acc_sh.at[dbp], sem_tp, add=True)

  pltpu.async_copy(einfo_hbm.at[wid, 0], eb0, sem_e0)
  pltpu.async_copy(w_hbm.at[wid, 0], wb0, sem_w0)
  pltpu.async_copy(einfo_hbm.at[wid, 1], eb1, sem_e1)
  pltpu.async_copy(w_hbm.at[wid, 1], wb1, sem_w1)
  pltpu.make_async_copy(einfo_hbm.at[wid, 0], eb0, sem_e0).wait()
  pltpu.async_copy(feat_hbm.at[eb0.at[0]], rw0, sem_g0)

  def _pair(it, _):
    c0 = 2 * it
    _half(c0, eb0, wb0, db0, rw0, sem_e0, sem_w0, sem_g0, sem_t0,
          eb1, db1, rw1, sem_e1, sem_g1, sem_t1)
    _half(c0 + 1, eb1, wb1, db1, rw1, sem_e1, sem_w1, sem_g1, sem_t1,
          eb0, db0, rw0, sem_e0, sem_g0, sem_t0)
    return 0
  lax.fori_loop(0, NCHUNK // 2, _pair, 0)

  pltpu.make_async_copy(rw0, acc_sh.at[db0], sem_t0).wait()
  pltpu.make_async_copy(rw1, acc_sh.at[db1], sem_t1).wait()

  plsc.subcore_barrier()

  def _wblk(j, _):
    b = j * NS + sid

    @pl.when(b < NRB)
    def _():
      @pl.when(cid == 0)
      def _():
        pltpu.sync_copy(acc_sh.at[pl.ds(b * RB, RB)],
                        out0_hbm.at[pl.ds(b * RB, RB)])

      @pl.when(cid == 1)
      def _():
        pltpu.sync_copy(acc_sh.at[pl.ds(b * RB, RB)],
                        out1_hbm.at[pl.ds(b * RB, RB)])
    return 0
  lax.fori_loop(0, RB_PER_TILE, _wblk, 0)


_spmm_sc = pl.kernel(
    _spmm_body,
    out_type=(jax.ShapeDtypeStruct((N, D), jnp.float32),
              jax.ShapeDtypeStruct((N, D), jnp.float32)),
    mesh=plsc.VectorSubcoreMesh(core_axis_name="c", subcore_axis_name="s",
                                num_cores=NC, num_subcores=NS),
    scratch_types=[
        pltpu.VMEM_SHARED((N, D), jnp.float32),
        pltpu.VMEM((2, C), jnp.int32),
        pltpu.VMEM((2, C), jnp.int32),
        pltpu.VMEM((C,), jnp.float32),
        pltpu.VMEM((C,), jnp.float32),
        pltpu.VMEM((C,), jnp.int32),
        pltpu.VMEM((C,), jnp.int32),
        pltpu.VMEM((C, D), jnp.float32),
        pltpu.VMEM((C, D), jnp.float32),
        pltpu.VMEM((RB, D), jnp.float32),
        pltpu.SemaphoreType.DMA,
        pltpu.SemaphoreType.DMA,
        pltpu.SemaphoreType.DMA,
        pltpu.SemaphoreType.DMA,
        pltpu.SemaphoreType.DMA,
        pltpu.SemaphoreType.DMA,
        pltpu.SemaphoreType.DMA,
        pltpu.SemaphoreType.DMA,
    ],
)


def _transform_body(x_ref, p0_ref, p1_ref, w0t_ref, w1t_ref, v0_ref, v1_ref,
                    out_ref):
  x = x_ref[...]
  h1 = p0_ref[...] + p1_ref[...]

  def _branch(h, wt_ref, v_ref):
    b = v_ref[0:1, :]
    scale = v_ref[1:2, :]
    offset = v_ref[2:3, :]
    y = jnp.maximum(
        jnp.dot(h, wt_ref[...], preferred_element_type=jnp.float32) + b, 0.0)
    mean = jnp.mean(y, axis=1, keepdims=True)
    yc = y - mean
    var = jnp.mean(yc * yc, axis=1, keepdims=True) + 1e-9
    return yc * scale * lax.rsqrt(var) + offset

  out_ref[...] = (_branch(x, w0t_ref, v0_ref) +
                  _branch(h1, w1t_ref, v1_ref))


def _transform_tc(feat_in, p0, p1, w0t, w1t, v0, v1):
  blk = 1000
  grid = N // blk
  return pl.pallas_call(
      _transform_body,
      out_shape=jax.ShapeDtypeStruct((N, D), jnp.float32),
      grid=(grid,),
      in_specs=[
          pl.BlockSpec((blk, D), lambda i: (i, 0)),
          pl.BlockSpec((blk, D), lambda i: (i, 0)),
          pl.BlockSpec((blk, D), lambda i: (i, 0)),
          pl.BlockSpec((D, D), lambda i: (0, 0)),
          pl.BlockSpec((D, D), lambda i: (0, 0)),
          pl.BlockSpec((8, D), lambda i: (0, 0)),
          pl.BlockSpec((8, D), lambda i: (0, 0)),
      ],
      out_specs=pl.BlockSpec((blk, D), lambda i: (i, 0)),
  )(feat_in, p0, p1, w0t, w1t, v0, v1)


@jax.jit
def kernel(feat_in, edge_index, edge_weight, W0, W1, b0, b1, scale0, scale1,
           offset0, offset1):
  dst = edge_index[0]
  src = edge_index[1]
  npad = E_PAD - E
  pad_idx = jnp.arange(npad, dtype=jnp.int32) % N
  src_p = jnp.concatenate([src, pad_idx])
  dst_p = jnp.concatenate([dst, pad_idx])
  w_p = jnp.concatenate([edge_weight, jnp.zeros((npad,), jnp.float32)])
  einfo = (jnp.stack([src_p, dst_p])
           .reshape(2, NW, NCHUNK, C).transpose(1, 2, 0, 3))
  wgt = w_p.reshape(NW, NCHUNK, C)

  p0, p1 = _spmm_sc(feat_in, einfo, wgt)

  def _pack(b, s, o):
    v = jnp.stack([b, s, o])
    return jnp.concatenate([v, jnp.zeros((5, D), jnp.float32)], axis=0)

  out = _transform_tc(feat_in, p0, p1, W0.T, W1.T,
                      _pack(b0, scale0, offset0), _pack(b1, scale1, offset1))
  return out


# speedup vs baseline: 11.0839x; 1.1418x over previous
import jax
import jax.numpy as jnp
from jax import lax
from jax.experimental import pallas as pl
from jax.experimental.pallas import tpu as pltpu
from jax.experimental.pallas import tpu_sc as plsc

N = 10000
E = 320000
D = 128

NC = 2
NS = 16
L = 16
NW = NC * NS
E_PER_W = E // NW
C = 80
NCHUNK = E_PER_W // C
R = 4
NITER = (NCHUNK + R - 1) // R
RB = 16
NRB = N // RB
RB_PER_TILE = (NRB + NS - 1) // NS

_GDN = lax.GatherDimensionNumbers(
    offset_dims=(), collapsed_slice_dims=(0,), start_index_map=(0,))


def _lane_bcast(v, j):
  idx = jnp.full((L, 1), j, jnp.int32)
  return lax.gather(v, idx, _GDN, (1,),
                    mode=lax.GatherScatterMode.PROMISE_IN_BOUNDS)


def _scale_rows(rw, wb, n_edges):
  def _group(g, _):
    wvec = wb[pl.ds(g * L, L)]
    for j in range(L):
      wj = _lane_bcast(wvec, j)
      e = g * L + j
      for k in range(D // L):
        rw[e, pl.ds(k * L, L)] = rw[e, pl.ds(k * L, L)] * wj
    return 0
  lax.fori_loop(0, n_edges // L, _group, 0)


def _spmm_body(feat_hbm, src_hbm, dst_hbm, w_hbm, out0_hbm, out1_hbm,
               acc_sh,
               sb0, sb1, sb2, sb3, db0, db1, db2, db3,
               wb0, wb1, wb2, wb3, rw0, rw1, rw2, rw3, zrow_v,
               sem_s0, sem_s1, sem_s2, sem_s3,
               sem_d0, sem_d1, sem_d2, sem_d3,
               sem_w0, sem_w1, sem_w2, sem_w3,
               sem_g0, sem_g1, sem_g2, sem_g3,
               sem_t0, sem_t1, sem_t2, sem_t3):
  cid = lax.axis_index("c")
  sid = lax.axis_index("s")
  wid = sid * NC + cid
  ebase = wid * E_PER_W

  sb = [sb0, sb1, sb2, sb3]
  db = [db0, db1, db2, db3]
  wb = [wb0, wb1, wb2, wb3]
  rw = [rw0, rw1, rw2, rw3]
  sem_s = [sem_s0, sem_s1, sem_s2, sem_s3]
  sem_d = [sem_d0, sem_d1, sem_d2, sem_d3]
  sem_w = [sem_w0, sem_w1, sem_w2, sem_w3]
  sem_g = [sem_g0, sem_g1, sem_g2, sem_g3]
  sem_t = [sem_t0, sem_t1, sem_t2, sem_t3]

  def _zlane(i, _):
    r = i // (D // L)
    k = i % (D // L)
    zrow_v[r, pl.ds(k * L, L)] = jnp.zeros((L,), jnp.float32)
    return 0
  lax.fori_loop(0, RB * (D // L), _zlane, 0)

  def _zblk(j, _):
    b = j * NS + sid

    @pl.when(b < NRB)
    def _():
      pltpu.sync_copy(zrow_v, acc_sh.at[pl.ds(b * RB, RB)])
    return 0
  lax.fori_loop(0, RB_PER_TILE, _zblk, 0)
  plsc.subcore_barrier()

  def _chunk(c, k):
    n = (k + 1) % R

    @pl.when(c + 1 < NCHUNK)
    def _():
      pltpu.make_async_copy(src_hbm.at[pl.ds(0, C)], sb[n], sem_s[n]).wait()

      @pl.when(c >= R - 1)
      def _():
        pltpu.make_async_copy(rw[n], acc_sh.at[db[n]], sem_t[n]).wait()
      pltpu.async_copy(feat_hbm.at[sb[n]], rw[n], sem_g[n])
      pltpu.async_copy(dst_hbm.at[pl.ds(ebase + (c + 1) * C, C)], db[n],
                       sem_d[n])
      pltpu.async_copy(w_hbm.at[pl.ds(ebase + (c + 1) * C, C)], wb[n],
                       sem_w[n])

    pltpu.make_async_copy(feat_hbm.at[sb[k]], rw[k], sem_g[k]).wait()

    @pl.when(c + R < NCHUNK)
    def _():
      pltpu.async_copy(src_hbm.at[pl.ds(ebase + (c + R) * C, C)], sb[k],
                       sem_s[k])

    pltpu.make_async_copy(w_hbm.at[pl.ds(0, C)], wb[k], sem_w[k]).wait()
    _scale_rows(rw[k], wb[k], C)

    pltpu.make_async_copy(dst_hbm.at[pl.ds(0, C)], db[k], sem_d[k]).wait()
    pltpu.async_copy(rw[k], acc_sh.at[db[k]], sem_t[k], add=True)

  for k in range(R):
    pltpu.async_copy(src_hbm.at[pl.ds(ebase + k * C, C)], sb[k], sem_s[k])
  pltpu.async_copy(dst_hbm.at[pl.ds(ebase, C)], db[0], sem_d[0])
  pltpu.async_copy(w_hbm.at[pl.ds(ebase, C)], wb[0], sem_w[0])
  pltpu.make_async_copy(src_hbm.at[pl.ds(0, C)], sb[0], sem_s[0]).wait()
  pltpu.async_copy(feat_hbm.at[sb[0]], rw[0], sem_g[0])

  def _ring(it, _):
    for k in range(R):
      c = R * it + k

      @pl.when(c < NCHUNK)
      def _():
        _chunk(c, k)
    return 0
  lax.fori_loop(0, NITER, _ring, 0)

  for k in range(R):
    pltpu.make_async_copy(rw[k], acc_sh.at[db[k]], sem_t[k]).wait()

  plsc.subcore_barrier()

  def _wblk(j, _):
    b = j * NS + sid

    @pl.when(b < NRB)
    def _():
      @pl.when(cid == 0)
      def _():
        pltpu.sync_copy(acc_sh.at[pl.ds(b * RB, RB)],
                        out0_hbm.at[pl.ds(b * RB, RB)])

      @pl.when(cid == 1)
      def _():
        pltpu.sync_copy(acc_sh.at[pl.ds(b * RB, RB)],
                        out1_hbm.at[pl.ds(b * RB, RB)])
    return 0
  lax.fori_loop(0, RB_PER_TILE, _wblk, 0)


_spmm_sc = pl.kernel(
    _spmm_body,
    out_type=(jax.ShapeDtypeStruct((N, D), jnp.float32),
              jax.ShapeDtypeStruct((N, D), jnp.float32)),
    mesh=plsc.VectorSubcoreMesh(core_axis_name="c", subcore_axis_name="s",
                                num_cores=NC, num_subcores=NS),
    scratch_types=(
        [pltpu.VMEM_SHARED((N, D), jnp.float32)]
        + [pltpu.VMEM((C,), jnp.int32) for _ in range(R)]
        + [pltpu.VMEM((C,), jnp.int32) for _ in range(R)]
        + [pltpu.VMEM((C,), jnp.float32) for _ in range(R)]
        + [pltpu.VMEM((C, D), jnp.float32) for _ in range(R)]
        + [pltpu.VMEM((RB, D), jnp.float32)]
        + [pltpu.SemaphoreType.DMA] * (5 * R)
    ),
)


def _transform_body(x_ref, p0_ref, p1_ref, w0t_ref, w1t_ref, v0_ref, v1_ref,
                    out_ref):
  x = x_ref[...]
  h1 = p0_ref[...] + p1_ref[...]

  def _branch(h, wt_ref, v_ref):
    b = v_ref[0:1, :]
    scale = v_ref[1:2, :]
    offset = v_ref[2:3, :]
    y = jnp.maximum(
        jnp.dot(h, wt_ref[...], preferred_element_type=jnp.float32) + b, 0.0)
    mean = jnp.mean(y, axis=1, keepdims=True)
    yc = y - mean
    var = jnp.mean(yc * yc, axis=1, keepdims=True) + 1e-9
    return yc * scale * lax.rsqrt(var) + offset

  out_ref[...] = (_branch(x, w0t_ref, v0_ref) +
                  _branch(h1, w1t_ref, v1_ref))


def _transform_tc(feat_in, p0, p1, w0t, w1t, v0, v1):
  blk = 1000
  grid = N // blk
  return pl.pallas_call(
      _transform_body,
      out_shape=jax.ShapeDtypeStruct((N, D), jnp.float32),
      grid=(grid,),
      in_specs=[
          pl.BlockSpec((blk, D), lambda i: (i, 0)),
          pl.BlockSpec((blk, D), lambda i: (i, 0)),
          pl.BlockSpec((blk, D), lambda i: (i, 0)),
          pl.BlockSpec((D, D), lambda i: (0, 0)),
          pl.BlockSpec((D, D), lambda i: (0, 0)),
          pl.BlockSpec((8, D), lambda i: (0, 0)),
          pl.BlockSpec((8, D), lambda i: (0, 0)),
      ],
      out_specs=pl.BlockSpec((blk, D), lambda i: (i, 0)),
  )(feat_in, p0, p1, w0t, w1t, v0, v1)


@jax.jit
def kernel(feat_in, edge_index, edge_weight, W0, W1, b0, b1, scale0, scale1,
           offset0, offset1):
  dst = edge_index[0]
  src = edge_index[1]

  p0, p1 = _spmm_sc(feat_in, src, dst, edge_weight)

  def _pack(b, s, o):
    v = jnp.stack([b, s, o])
    return jnp.concatenate([v, jnp.zeros((5, D), jnp.float32)], axis=0)

  out = _transform_tc(feat_in, p0, p1, W0.T, W1.T,
                      _pack(b0, scale0, offset0), _pack(b1, scale1, offset1))
  return out


# speedup vs baseline: 11.1274x; 1.0039x over previous
import jax
import jax.numpy as jnp
from jax import lax
from jax.experimental import pallas as pl
from jax.experimental.pallas import tpu as pltpu
from jax.experimental.pallas import tpu_sc as plsc

N = 10000
E = 320000
D = 128

NC = 2
NS = 16
L = 16
NW = NC * NS
E_PER_W = E // NW
C = 80
NCHUNK = E_PER_W // C
R = 4
NITER = (NCHUNK + R - 1) // R
RB = 16
NRB = N // RB
RB_PER_TILE = (NRB + NS - 1) // NS

_GDN = lax.GatherDimensionNumbers(
    offset_dims=(), collapsed_slice_dims=(0,), start_index_map=(0,))


def _lane_bcast(v, j):
  idx = jnp.full((L, 1), j, jnp.int32)
  return lax.gather(v, idx, _GDN, (1,),
                    mode=lax.GatherScatterMode.PROMISE_IN_BOUNDS)


def _scale_rows(rw, wb, n_edges):
  def _group(g, _):
    wvec = wb[pl.ds(g * L, L)]
    for j in range(L):
      wj = _lane_bcast(wvec, j)
      e = g * L + j
      for k in range(D // L):
        rw[e, pl.ds(k * L, L)] = rw[e, pl.ds(k * L, L)] * wj
    return 0
  lax.fori_loop(0, n_edges // L, _group, 0)


def _spmm_body(feat_hbm, src_hbm, dst_hbm, w_hbm, out0_hbm, out1_hbm,
               acc_sh,
               sb0, sb1, sb2, sb3, db0, db1, db2, db3,
               wb0, wb1, wb2, wb3, rw0, rw1, rw2, rw3, zrow_v,
               sem_s0, sem_s1, sem_s2, sem_s3,
               sem_d0, sem_d1, sem_d2, sem_d3,
               sem_w0, sem_w1, sem_w2, sem_w3,
               sem_g0, sem_g1, sem_g2, sem_g3,
               sem_t0, sem_t1, sem_t2, sem_t3):
  cid = lax.axis_index("c")
  sid = lax.axis_index("s")
  wid = sid * NC + cid
  ebase = wid * E_PER_W

  sb = [sb0, sb1, sb2, sb3]
  db = [db0, db1, db2, db3]
  wb = [wb0, wb1, wb2, wb3]
  rw = [rw0, rw1, rw2, rw3]
  sem_s = [sem_s0, sem_s1, sem_s2, sem_s3]
  sem_d = [sem_d0, sem_d1, sem_d2, sem_d3]
  sem_w = [sem_w0, sem_w1, sem_w2, sem_w3]
  sem_g = [sem_g0, sem_g1, sem_g2, sem_g3]
  sem_t = [sem_t0, sem_t1, sem_t2, sem_t3]

  def _zlane(i, _):
    r = i // (D // L)
    k = i % (D // L)
    zrow_v[r, pl.ds(k * L, L)] = jnp.zeros((L,), jnp.float32)
    return 0
  lax.fori_loop(0, RB * (D // L), _zlane, 0)

  def _zblk(j, _):
    b = j * NS + sid

    @pl.when(b < NRB)
    def _():
      pltpu.sync_copy(zrow_v, acc_sh.at[pl.ds(b * RB, RB)])
    return 0
  lax.fori_loop(0, RB_PER_TILE, _zblk, 0)
  plsc.subcore_barrier()

  def _chunk(c, k):
    n = (k + 1) % R

    @pl.when(c + 1 < NCHUNK)
    def _():
      pltpu.make_async_copy(src_hbm.at[pl.ds(0, C)], sb[n], sem_s[n]).wait()

      @pl.when(c >= R - 1)
      def _():
        pltpu.make_async_copy(rw[n], acc_sh.at[db[n]], sem_t[n]).wait()
      pltpu.async_copy(feat_hbm.at[sb[n]], rw[n], sem_g[n])
      pltpu.async_copy(dst_hbm.at[pl.ds(ebase + (c + 1) * C, C)], db[n],
                       sem_d[n])
      pltpu.async_copy(w_hbm.at[pl.ds(ebase + (c + 1) * C, C)], wb[n],
                       sem_w[n])

    pltpu.make_async_copy(feat_hbm.at[sb[k]], rw[k], sem_g[k]).wait()

    @pl.when(c + R < NCHUNK)
    def _():
      pltpu.async_copy(src_hbm.at[pl.ds(ebase + (c + R) * C, C)], sb[k],
                       sem_s[k])

    pltpu.make_async_copy(w_hbm.at[pl.ds(0, C)], wb[k], sem_w[k]).wait()
    _scale_rows(rw[k], wb[k], C)

    pltpu.make_async_copy(dst_hbm.at[pl.ds(0, C)], db[k], sem_d[k]).wait()
    pltpu.async_copy(rw[k], acc_sh.at[db[k]], sem_t[k], add=True)

  for k in range(R):
    pltpu.async_copy(src_hbm.at[pl.ds(ebase + k * C, C)], sb[k], sem_s[k])
  pltpu.async_copy(dst_hbm.at[pl.ds(ebase, C)], db[0], sem_d[0])
  pltpu.async_copy(w_hbm.at[pl.ds(ebase, C)], wb[0], sem_w[0])
  pltpu.make_async_copy(src_hbm.at[pl.ds(0, C)], sb[0], sem_s[0]).wait()
  pltpu.async_copy(feat_hbm.at[sb[0]], rw[0], sem_g[0])

  def _ring(it, _):
    for k in range(R):
      c = R * it + k

      @pl.when(c < NCHUNK)
      def _():
        _chunk(c, k)
    return 0
  lax.fori_loop(0, NITER, _ring, 0)

  for k in range(R):
    pltpu.make_async_copy(rw[k], acc_sh.at[db[k]], sem_t[k]).wait()

  plsc.subcore_barrier()

  def _wblk(j, _):
    b = j * NS + sid

    @pl.when(b < NRB)
    def _():
      @pl.when(cid == 0)
      def _():
        pltpu.sync_copy(acc_sh.at[pl.ds(b * RB, RB)],
                        out0_hbm.at[pl.ds(b * RB, RB)])

      @pl.when(cid == 1)
      def _():
        pltpu.sync_copy(acc_sh.at[pl.ds(b * RB, RB)],
                        out1_hbm.at[pl.ds(b * RB, RB)])
    return 0
  lax.fori_loop(0, RB_PER_TILE, _wblk, 0)


_spmm_sc = pl.kernel(
    _spmm_body,
    out_type=(jax.ShapeDtypeStruct((N, D), jnp.float32),
              jax.ShapeDtypeStruct((N, D), jnp.float32)),
    mesh=plsc.VectorSubcoreMesh(core_axis_name="c", subcore_axis_name="s",
                                num_cores=NC, num_subcores=NS),
    scratch_types=(
        [pltpu.VMEM_SHARED((N, D), jnp.float32)]
        + [pltpu.VMEM((C,), jnp.int32) for _ in range(R)]
        + [pltpu.VMEM((C,), jnp.int32) for _ in range(R)]
        + [pltpu.VMEM((C,), jnp.float32) for _ in range(R)]
        + [pltpu.VMEM((C, D), jnp.float32) for _ in range(R)]
        + [pltpu.VMEM((RB, D), jnp.float32)]
        + [pltpu.SemaphoreType.DMA] * (5 * R)
    ),
)


def _ln_branch(h, wt_ref, v_ref):
  b = v_ref[0:1, :]
  scale = v_ref[1:2, :]
  offset = v_ref[2:3, :]
  y = jnp.maximum(
      jnp.dot(h, wt_ref[...], preferred_element_type=jnp.float32) + b, 0.0)
  mean = jnp.mean(y, axis=1, keepdims=True)
  yc = y - mean
  var = jnp.mean(yc * yc, axis=1, keepdims=True) + 1e-9
  return yc * scale * lax.rsqrt(var) + offset


def _branch0_body(x_ref, w0t_ref, v0_ref, out_ref):
  out_ref[...] = _ln_branch(x_ref[...], w0t_ref, v0_ref)


def _branch1_body(y0_ref, p0_ref, p1_ref, w1t_ref, v1_ref, out_ref):
  h1 = p0_ref[...] + p1_ref[...]
  out_ref[...] = y0_ref[...] + _ln_branch(h1, w1t_ref, v1_ref)


_BLK = 1000


def _row_spec():
  return pl.BlockSpec((_BLK, D), lambda i: (i, 0))


def _full_spec(rows):
  return pl.BlockSpec((rows, D), lambda i: (0, 0))


def _branch0_tc(feat_in, w0t, v0):
  return pl.pallas_call(
      _branch0_body,
      out_shape=jax.ShapeDtypeStruct((N, D), jnp.float32),
      grid=(N // _BLK,),
      in_specs=[_row_spec(), _full_spec(D), _full_spec(8)],
      out_specs=_row_spec(),
  )(feat_in, w0t, v0)


def _branch1_tc(y0, p0, p1, w1t, v1):
  return pl.pallas_call(
      _branch1_body,
      out_shape=jax.ShapeDtypeStruct((N, D), jnp.float32),
      grid=(N // _BLK,),
      in_specs=[_row_spec(), _row_spec(), _row_spec(), _full_spec(D),
                _full_spec(8)],
      out_specs=_row_spec(),
  )(y0, p0, p1, w1t, v1)


@jax.jit
def kernel(feat_in, edge_index, edge_weight, W0, W1, b0, b1, scale0, scale1,
           offset0, offset1):
  dst = edge_index[0]
  src = edge_index[1]

  p0, p1 = _spmm_sc(feat_in, src, dst, edge_weight)

  def _pack(b, s, o):
    v = jnp.stack([b, s, o])
    return jnp.concatenate([v, jnp.zeros((5, D), jnp.float32)], axis=0)

  y0 = _branch0_tc(feat_in, W0.T, _pack(b0, scale0, offset0))
  out = _branch1_tc(y0, p0, p1, W1.T, _pack(b1, scale1, offset1))
  return out
